# baseline (speedup 1.0000x reference)
import functools

import jax
import jax.numpy as jnp
from jax import lax
from jax.experimental import pallas as pl
from jax.experimental.pallas import tpu as pltpu
from jax.experimental.pallas import tpu_sc as plsc

N = 10000
E = 320000
D = 128
DP = 144
NW = 32
CHUNK = 80
A = 180
B = 75
T_CHUNKS = 16 * (A + B)
E_PAD = T_CHUNKS * CHUNK
ACC_ROWS = 10240
ROWS_PT = ACC_ROWS // 16
TRASH = N
KR = 3
KI = 5

_mesh = plsc.VectorSubcoreMesh(core_axis_name="c", subcore_axis_name="s")


@functools.partial(
    pl.kernel,
    mesh=_mesh,
    compiler_params=pltpu.CompilerParams(use_tc_tiling_on_sc=False),
    out_type=jax.ShapeDtypeStruct((2, ACC_ROWS, DP), jnp.float32),
    scratch_types=[
        [pltpu.VMEM((2, CHUNK), jnp.int32)] * KI,
        [pltpu.VMEM((CHUNK, DP), jnp.float32)] * KR,
        pltpu.VMEM_SHARED((ACC_ROWS, DP), jnp.float32),
        [pltpu.SemaphoreType.DMA] * KI,
        [pltpu.SemaphoreType.DMA] * KR,
        [pltpu.SemaphoreType.DMA] * KR,
    ],
)
def _sc_accumulate(xext, exi, out, idxu, rows, acc, isem, gsem, ssem):
    c = lax.axis_index("c")
    s = lax.axis_index("s")

    zv = jnp.zeros((16,), jnp.float32)

    def zrow(i, carry):
        for j in range(DP // 16):
            rows[0][i, pl.ds(j * 16, 16)] = zv
        return carry

    lax.fori_loop(0, CHUNK, zrow, 0)
    for j in range(ROWS_PT // CHUNK):
        pltpu.sync_copy(rows[0], acc.at[pl.ds(s * ROWS_PT + j * CHUNK, CHUNK)])
    plsc.subcore_barrier()

    def pipeline(start, n):
        def ifetch(t, bi):
            return pltpu.make_async_copy(exi.at[start + t], idxu[bi],
                                         isem[bi])

        def gather(t, br, bi):
            return pltpu.make_async_copy(xext.at[idxu[bi].at[0]], rows[br],
                                         gsem[br])

        def scatter(t, br, bi):
            return pltpu.make_async_copy(rows[br], acc.at[idxu[bi].at[1]],
                                         ssem[br])

        for t in range(4):
            ifetch(t, t).start()
        for t in range(2):
            ifetch(t, t).wait()
            gather(t, t, t).start()

        def body(jj, carry):
            for u in range(KR * KI):
                j = jj * (KR * KI) + u
                br = u % KR
                bi = u % KI
                gather(j, br, bi).wait()
                scatter(j, br, bi).start(add=True)

                @pl.when(j >= 1)
                def _():
                    scatter(j - 1, (u - 1) % KR, (u - 1) % KI).wait()

                    @pl.when(j + 3 < n)
                    def _():
                        ifetch(j + 3, (u + 3) % KI).start()

                @pl.when(j + 2 < n)
                def _():
                    ifetch(j + 2, (u + 2) % KI).wait()
                    gather(j + 2, (u + 2) % KR, (u + 2) % KI).start()

            return carry

        lax.fori_loop(0, n // (KR * KI), body, 0)
        scatter(n - 1, (n - 1) % KR, (n - 1) % KI).wait()

    @pl.when(c == 0)
    def _():
        pipeline(s * A, A)

    @pl.when(c == 1)
    def _():
        pipeline(16 * A + s * B, B)

    plsc.subcore_barrier()

    for j in range(ROWS_PT // CHUNK):
        off = s * ROWS_PT + j * CHUNK
        pltpu.sync_copy(acc.at[pl.ds(off, CHUNK)], rows[0])
        pltpu.sync_copy(rows[0], out.at[c, pl.ds(off, CHUNK)])


BN = 400


def _finish_body(p_ref, x_ref, o_ref):
    p0 = p_ref[0]
    p1 = p_ref[1]
    cnt = p0[:, D:D + 1] + p1[:, D:D + 1]
    sums = p0[:, :D] + p1[:, :D]
    o_ref[:, :D] = sums / jnp.maximum(cnt, 1.0)
    o_ref[:, D:] = jnp.where(cnt > 0.0, x_ref[...], 0.0)


_finish = pl.pallas_call(
    _finish_body,
    grid=(N // BN,),
    in_specs=[
        pl.BlockSpec((2, BN, DP), lambda i: (0, i, 0)),
        pl.BlockSpec((BN, D), lambda i: (i, 0)),
    ],
    out_specs=pl.BlockSpec((BN, 2 * D), lambda i: (i, 0)),
    out_shape=jax.ShapeDtypeStruct((N, 2 * D), jnp.float32),
)


@jax.jit
def kernel(x, es):
    col = es[0]
    row = es[1]
    xext = jnp.concatenate(
        [x,
         jnp.ones((N, 1), jnp.float32),
         jnp.zeros((N, DP - D - 1), jnp.float32)], axis=1)
    pad = E_PAD - E
    rowp = jnp.concatenate([row, jnp.zeros((pad,), jnp.int32)])
    colp = jnp.concatenate([col, jnp.full((pad,), TRASH, jnp.int32)])
    exi = jnp.concatenate(
        [rowp.reshape(T_CHUNKS, 1, CHUNK), colp.reshape(T_CHUNKS, 1, CHUNK)],
        axis=1)
    partial = _sc_accumulate(xext, exi)
    return _finish(partial, x)


# speedup vs baseline: 1.8904x; 1.8904x over previous
import functools

import jax
import jax.numpy as jnp
from jax import lax
from jax.experimental import pallas as pl
from jax.experimental.pallas import tpu as pltpu
from jax.experimental.pallas import tpu_sc as plsc

N = 10000
E = 320000
D = 128
DP = 144
NW = 32
CHUNK = 48
A = 309
B = 108
T_CHUNKS = 16 * (A + B)
E_PAD = T_CHUNKS * CHUNK
ACC_ROWS = 10240
ROWS_PT = ACC_ROWS // 16
TRASH = N
K = 3

_mesh = plsc.VectorSubcoreMesh(core_axis_name="c", subcore_axis_name="s")


@functools.partial(
    pl.kernel,
    mesh=_mesh,
    compiler_params=pltpu.CompilerParams(use_tc_tiling_on_sc=False),
    out_type=jax.ShapeDtypeStruct((2, ACC_ROWS, DP), jnp.float32),
    scratch_types=[
        pltpu.VMEM((max(A, B), CHUNK), jnp.int32),
        [pltpu.VMEM((2, CHUNK), jnp.int32)] * K,
        [pltpu.VMEM((CHUNK, DP), jnp.float32)] * K,
        pltpu.VMEM_SHARED((ACC_ROWS, DP), jnp.float32),
        [pltpu.SemaphoreType.DMA] * K,
        [pltpu.SemaphoreType.DMA] * K,
    ],
)
def _sc_accumulate(xext, exi, out, packed, idxu, rows, acc, gsem, ssem):
    c = lax.axis_index("c")
    s = lax.axis_index("s")

    zv = jnp.zeros((16,), jnp.float32)

    def zrow(i, carry):
        for j in range(DP // 16):
            rows[0][i, pl.ds(j * 16, 16)] = zv
        return carry

    lax.fori_loop(0, CHUNK, zrow, 0)
    for j in range(ROWS_PT // CHUNK):
        pltpu.sync_copy(rows[0], acc.at[pl.ds(s * ROWS_PT + j * CHUNK, CHUNK)])
    rem = ROWS_PT - (ROWS_PT // CHUNK) * CHUNK
    pltpu.sync_copy(
        rows[0].at[pl.ds(0, rem)],
        acc.at[pl.ds(s * ROWS_PT + ROWS_PT - rem, rem)])
    plsc.subcore_barrier()

    def unpack(t, b):
        for k in range(CHUNK // 16):
            v = packed[t, pl.ds(k * 16, 16)]
            idxu[b][0, pl.ds(k * 16, 16)] = v & 0xFFFF
            idxu[b][1, pl.ds(k * 16, 16)] = v >> 16

    def gather(t, b):
        return pltpu.make_async_copy(xext.at[idxu[b].at[0]], rows[b], gsem[b])

    def scatter(t, b):
        return pltpu.make_async_copy(rows[b], acc.at[idxu[b].at[1]], ssem[b])

    def pipeline(start, n):
        pltpu.sync_copy(exi.at[pl.ds(start, n)], packed.at[pl.ds(0, n)])
        for t in range(2):
            unpack(t, t)
            gather(t, t).start()

        def body(jj, carry):
            for u in range(K):
                b = u % K
                j = jj * K + u
                gather(j, b).wait()
                scatter(j, b).start(add=True)
                b2 = (u + 2) % K

                @pl.when(j + 2 < n)
                def _():
                    @pl.when(j >= 1)
                    def _():
                        scatter(j - 1, b2).wait()
                    unpack(j + 2, b2)
                    gather(j + 2, b2).start()

            return carry

        lax.fori_loop(0, n // K, body, 0)
        for u in range(K):
            j = n - K + u
            scatter(j, j % K).wait()

    @pl.when(c == 0)
    def _():
        pipeline(s * A, A)

    @pl.when(c == 1)
    def _():
        pipeline(16 * A + s * B, B)

    plsc.subcore_barrier()

    for j in range(ROWS_PT // CHUNK):
        off = s * ROWS_PT + j * CHUNK
        pltpu.sync_copy(acc.at[pl.ds(off, CHUNK)], rows[0])
        pltpu.sync_copy(rows[0], out.at[c, pl.ds(off, CHUNK)])
    off = s * ROWS_PT + ROWS_PT - rem
    pltpu.sync_copy(acc.at[pl.ds(off, rem)], rows[0].at[pl.ds(0, rem)])
    pltpu.sync_copy(rows[0].at[pl.ds(0, rem)], out.at[c, pl.ds(off, rem)])


BN = 400


def _finish_body(p_ref, x_ref, o_ref):
    p0 = p_ref[0]
    p1 = p_ref[1]
    cnt = p0[:, D:D + 1] + p1[:, D:D + 1]
    sums = p0[:, :D] + p1[:, :D]
    o_ref[:, :D] = sums / jnp.maximum(cnt, 1.0)
    o_ref[:, D:] = jnp.where(cnt > 0.0, x_ref[...], 0.0)


_finish = pl.pallas_call(
    _finish_body,
    grid=(N // BN,),
    in_specs=[
        pl.BlockSpec((2, BN, DP), lambda i: (0, i, 0)),
        pl.BlockSpec((BN, D), lambda i: (i, 0)),
    ],
    out_specs=pl.BlockSpec((BN, 2 * D), lambda i: (i, 0)),
    out_shape=jax.ShapeDtypeStruct((N, 2 * D), jnp.float32),
)


@jax.jit
def kernel(x, es):
    col = es[0]
    row = es[1]
    xext = jnp.concatenate(
        [x,
         jnp.ones((N, 1), jnp.float32),
         jnp.zeros((N, DP - D - 1), jnp.float32)], axis=1)
    pad = E_PAD - E
    rowp = jnp.concatenate([row, jnp.zeros((pad,), jnp.int32)])
    colp = jnp.concatenate([col, jnp.full((pad,), TRASH, jnp.int32)])
    packed = jnp.bitwise_or(jnp.left_shift(colp, 16), rowp)
    partial = _sc_accumulate(xext, packed.reshape(T_CHUNKS, CHUNK))
    return _finish(partial, x)


# speedup vs baseline: 2.1621x; 1.1437x over previous
import functools

import jax
import jax.numpy as jnp
from jax import lax
from jax.experimental import pallas as pl
from jax.experimental.pallas import tpu as pltpu
from jax.experimental.pallas import tpu_sc as plsc

N = 10000
E = 320000
D = 128
DP = 144
NW = 32
CHUNK = 48
A = 237
B = 180
T_CHUNKS = 16 * (A + B)
E_PAD = T_CHUNKS * CHUNK
ACC_ROWS = 10240
ROWS_PT = ACC_ROWS // 16
TRASH = N
K = 3

_mesh = plsc.VectorSubcoreMesh(core_axis_name="c", subcore_axis_name="s")


@functools.partial(
    pl.kernel,
    mesh=_mesh,
    compiler_params=pltpu.CompilerParams(use_tc_tiling_on_sc=False),
    out_type=jax.ShapeDtypeStruct((2, ACC_ROWS, DP), jnp.float32),
    scratch_types=[
        pltpu.VMEM((max(A, B), CHUNK), jnp.int32),
        [pltpu.VMEM((2, CHUNK), jnp.int32)] * K,
        [pltpu.VMEM((CHUNK, DP), jnp.float32)] * K,
        pltpu.VMEM_SHARED((ACC_ROWS, DP), jnp.float32),
        [pltpu.SemaphoreType.DMA] * K,
        [pltpu.SemaphoreType.DMA] * K,
    ],
)
def _sc_accumulate(xext, exi, out, packed, idxu, rows, acc, gsem, ssem):
    c = lax.axis_index("c")
    s = lax.axis_index("s")

    zv = jnp.zeros((16,), jnp.float32)

    def zrow(i, carry):
        for j in range(DP // 16):
            rows[0][i, pl.ds(j * 16, 16)] = zv
        return carry

    lax.fori_loop(0, CHUNK, zrow, 0)
    for j in range(ROWS_PT // CHUNK):
        pltpu.sync_copy(rows[0], acc.at[pl.ds(s * ROWS_PT + j * CHUNK, CHUNK)])
    rem = ROWS_PT - (ROWS_PT // CHUNK) * CHUNK
    pltpu.sync_copy(
        rows[0].at[pl.ds(0, rem)],
        acc.at[pl.ds(s * ROWS_PT + ROWS_PT - rem, rem)])
    plsc.subcore_barrier()

    def unpack(t, b):
        for k in range(CHUNK // 16):
            v = packed[t, pl.ds(k * 16, 16)]
            idxu[b][0, pl.ds(k * 16, 16)] = v & 0xFFFF
            idxu[b][1, pl.ds(k * 16, 16)] = v >> 16

    def gather(t, b):
        return pltpu.make_async_copy(xext.at[idxu[b].at[0]], rows[b], gsem[b])

    def scatter(t, b):
        return pltpu.make_async_copy(rows[b], acc.at[idxu[b].at[1]], ssem[b])

    def pipeline(start, n):
        pltpu.sync_copy(exi.at[pl.ds(start, n)], packed.at[pl.ds(0, n)])
        for t in range(2):
            unpack(t, t)
            gather(t, t).start()

        def body(jj, carry):
            for u in range(K):
                b = u % K
                j = jj * K + u
                gather(j, b).wait()
                scatter(j, b).start(add=True)
                b2 = (u + 2) % K

                @pl.when(j + 2 < n)
                def _():
                    @pl.when(j >= 1)
                    def _():
                        scatter(j - 1, b2).wait()
                    unpack(j + 2, b2)
                    gather(j + 2, b2).start()

            return carry

        lax.fori_loop(0, n // K, body, 0)
        for u in range(K):
            j = n - K + u
            scatter(j, j % K).wait()

    @pl.when(c == 0)
    def _():
        pipeline(s * A, A)

    @pl.when(c == 1)
    def _():
        pipeline(16 * A + s * B, B)

    plsc.subcore_barrier()

    for j in range(ROWS_PT // CHUNK):
        off = s * ROWS_PT + j * CHUNK
        pltpu.sync_copy(acc.at[pl.ds(off, CHUNK)], rows[0])
        pltpu.sync_copy(rows[0], out.at[c, pl.ds(off, CHUNK)])
    off = s * ROWS_PT + ROWS_PT - rem
    pltpu.sync_copy(acc.at[pl.ds(off, rem)], rows[0].at[pl.ds(0, rem)])
    pltpu.sync_copy(rows[0].at[pl.ds(0, rem)], out.at[c, pl.ds(off, rem)])


BN = 400


def _finish_body(p_ref, x_ref, o_ref):
    p0 = p_ref[0]
    p1 = p_ref[1]
    cnt = p0[:, D:D + 1] + p1[:, D:D + 1]
    sums = p0[:, :D] + p1[:, :D]
    o_ref[:, :D] = sums / jnp.maximum(cnt, 1.0)
    o_ref[:, D:] = jnp.where(cnt > 0.0, x_ref[...], 0.0)


_finish = pl.pallas_call(
    _finish_body,
    grid=(N // BN,),
    in_specs=[
        pl.BlockSpec((2, BN, DP), lambda i: (0, i, 0)),
        pl.BlockSpec((BN, D), lambda i: (i, 0)),
    ],
    out_specs=pl.BlockSpec((BN, 2 * D), lambda i: (i, 0)),
    out_shape=jax.ShapeDtypeStruct((N, 2 * D), jnp.float32),
)


@jax.jit
def kernel(x, es):
    col = es[0]
    row = es[1]
    xext = jnp.concatenate(
        [x,
         jnp.ones((N, 1), jnp.float32),
         jnp.zeros((N, DP - D - 1), jnp.float32)], axis=1)
    pad = E_PAD - E
    rowp = jnp.concatenate([row, jnp.zeros((pad,), jnp.int32)])
    colp = jnp.concatenate([col, jnp.full((pad,), TRASH, jnp.int32)])
    packed = jnp.bitwise_or(jnp.left_shift(colp, 16), rowp)
    partial = _sc_accumulate(xext, packed.reshape(T_CHUNKS, CHUNK))
    return _finish(partial, x)


# speedup vs baseline: 2.2847x; 1.0567x over previous
import functools

import jax
import jax.numpy as jnp
from jax import lax
from jax.experimental import pallas as pl
from jax.experimental.pallas import tpu as pltpu
from jax.experimental.pallas import tpu_sc as plsc

N = 10000
E = 320000
D = 128
DP = 144
NW = 32
CHUNK = 48
A = 216
B = 201
T_CHUNKS = 16 * (A + B)
E_PAD = T_CHUNKS * CHUNK
ACC_ROWS = 10240
ROWS_PT = ACC_ROWS // 16
TRASH = N
K = 3

_mesh = plsc.VectorSubcoreMesh(core_axis_name="c", subcore_axis_name="s")


@functools.partial(
    pl.kernel,
    mesh=_mesh,
    compiler_params=pltpu.CompilerParams(use_tc_tiling_on_sc=False),
    out_type=jax.ShapeDtypeStruct((2, ACC_ROWS, DP), jnp.float32),
    scratch_types=[
        pltpu.VMEM((max(A, B), CHUNK), jnp.int32),
        [pltpu.VMEM((2, CHUNK), jnp.int32)] * K,
        [pltpu.VMEM((CHUNK, DP), jnp.float32)] * K,
        pltpu.VMEM_SHARED((ACC_ROWS, DP), jnp.float32),
        [pltpu.SemaphoreType.DMA] * K,
        [pltpu.SemaphoreType.DMA] * K,
    ],
)
def _sc_accumulate(xext, exi, out, packed, idxu, rows, acc, gsem, ssem):
    c = lax.axis_index("c")
    s = lax.axis_index("s")

    zv = jnp.zeros((16,), jnp.float32)

    def zrow(i, carry):
        for j in range(DP // 16):
            rows[0][i, pl.ds(j * 16, 16)] = zv
        return carry

    lax.fori_loop(0, CHUNK, zrow, 0)
    for j in range(ROWS_PT // CHUNK):
        pltpu.sync_copy(rows[0], acc.at[pl.ds(s * ROWS_PT + j * CHUNK, CHUNK)])
    rem = ROWS_PT - (ROWS_PT // CHUNK) * CHUNK
    pltpu.sync_copy(
        rows[0].at[pl.ds(0, rem)],
        acc.at[pl.ds(s * ROWS_PT + ROWS_PT - rem, rem)])
    plsc.subcore_barrier()

    def unpack(t, b):
        for k in range(CHUNK // 16):
            v = packed[t, pl.ds(k * 16, 16)]
            idxu[b][0, pl.ds(k * 16, 16)] = v & 0xFFFF
            idxu[b][1, pl.ds(k * 16, 16)] = v >> 16

    def gather(t, b):
        return pltpu.make_async_copy(xext.at[idxu[b].at[0]], rows[b], gsem[b])

    def scatter(t, b):
        return pltpu.make_async_copy(rows[b], acc.at[idxu[b].at[1]], ssem[b])

    def pipeline(start, n):
        pltpu.sync_copy(exi.at[pl.ds(start, n)], packed.at[pl.ds(0, n)])
        for t in range(2):
            unpack(t, t)
            gather(t, t).start()

        def body(jj, carry):
            for u in range(K):
                b = u % K
                j = jj * K + u
                gather(j, b).wait()
                scatter(j, b).start(add=True)
                b2 = (u + 2) % K

                @pl.when(j + 2 < n)
                def _():
                    @pl.when(j >= 1)
                    def _():
                        scatter(j - 1, b2).wait()
                    unpack(j + 2, b2)
                    gather(j + 2, b2).start()

            return carry

        lax.fori_loop(0, n // K, body, 0)
        for u in range(K):
            j = n - K + u
            scatter(j, j % K).wait()

    @pl.when(c == 0)
    def _():
        pipeline(s * A, A)

    @pl.when(c == 1)
    def _():
        pipeline(16 * A + s * B, B)

    plsc.subcore_barrier()

    off = s * ROWS_PT
    pltpu.sync_copy(acc.at[pl.ds(off, ROWS_PT)], out.at[c, pl.ds(off, ROWS_PT)])


BN = 400


def _finish_body(p_ref, x_ref, o_ref):
    p0 = p_ref[0]
    p1 = p_ref[1]
    cnt = p0[:, D:D + 1] + p1[:, D:D + 1]
    sums = p0[:, :D] + p1[:, :D]
    o_ref[:, :D] = sums / jnp.maximum(cnt, 1.0)
    o_ref[:, D:] = jnp.where(cnt > 0.0, x_ref[...], 0.0)


_finish = pl.pallas_call(
    _finish_body,
    grid=(N // BN,),
    in_specs=[
        pl.BlockSpec((2, BN, DP), lambda i: (0, i, 0)),
        pl.BlockSpec((BN, D), lambda i: (i, 0)),
    ],
    out_specs=pl.BlockSpec((BN, 2 * D), lambda i: (i, 0)),
    out_shape=jax.ShapeDtypeStruct((N, 2 * D), jnp.float32),
)


@jax.jit
def kernel(x, es):
    col = es[0]
    row = es[1]
    xext = jnp.concatenate(
        [x,
         jnp.ones((N, 1), jnp.float32),
         jnp.zeros((N, DP - D - 1), jnp.float32)], axis=1)
    pad = E_PAD - E
    rowp = jnp.concatenate([row, jnp.zeros((pad,), jnp.int32)])
    colp = jnp.concatenate([col, jnp.full((pad,), TRASH, jnp.int32)])
    packed = jnp.bitwise_or(jnp.left_shift(colp, 16), rowp)
    partial = _sc_accumulate(xext, packed.reshape(T_CHUNKS, CHUNK))
    return _finish(partial, x)


# speedup vs baseline: 2.3051x; 1.0089x over previous
import functools

import jax
import jax.numpy as jnp
from jax import lax
from jax.experimental import pallas as pl
from jax.experimental.pallas import tpu as pltpu
from jax.experimental.pallas import tpu_sc as plsc

N = 10000
E = 320000
D = 128
DP = 144
NW = 32
CHUNK = 48
A = 216
B = 201
T_CHUNKS = 16 * (A + B)
E_PAD = T_CHUNKS * CHUNK
ACC_ROWS = 10240
ROWS_PT = ACC_ROWS // 16
TRASH = N
K = 3

_mesh = plsc.VectorSubcoreMesh(core_axis_name="c", subcore_axis_name="s")


@functools.partial(
    pl.kernel,
    mesh=_mesh,
    compiler_params=pltpu.CompilerParams(use_tc_tiling_on_sc=False),
    out_type=jax.ShapeDtypeStruct((2, ACC_ROWS, DP), jnp.float32),
    scratch_types=[
        pltpu.VMEM((max(A, B), CHUNK), jnp.int32),
        [pltpu.VMEM((2, CHUNK), jnp.int32)] * K,
        [pltpu.VMEM((CHUNK, DP), jnp.float32)] * K,
        pltpu.VMEM_SHARED((ACC_ROWS, DP), jnp.float32),
        [pltpu.SemaphoreType.DMA] * K,
        [pltpu.SemaphoreType.DMA] * K,
        pltpu.SemaphoreType.DMA,
    ],
)
def _sc_accumulate(xext, exi, out, packed, idxu, rows, acc, gsem, ssem, zsem):
    c = lax.axis_index("c")
    s = lax.axis_index("s")

    def unpack(t, b):
        for k in range(CHUNK // 16):
            v = packed[t, pl.ds(k * 16, 16)]
            idxu[b][0, pl.ds(k * 16, 16)] = v & 0xFFFF
            idxu[b][1, pl.ds(k * 16, 16)] = v >> 16

    def gather(t, b):
        return pltpu.make_async_copy(xext.at[idxu[b].at[0]], rows[b], gsem[b])

    def scatter(t, b):
        return pltpu.make_async_copy(rows[b], acc.at[idxu[b].at[1]], ssem[b])

    def prologue(start, n):
        pltpu.sync_copy(exi.at[pl.ds(start, n)], packed.at[pl.ds(0, n)])
        for t in range(2):
            unpack(t, t)
            gather(t, t).start()

    @pl.when(c == 0)
    def _():
        prologue(s * A, A)

    @pl.when(c == 1)
    def _():
        prologue(16 * A + s * B, B)

    zv = jnp.zeros((16,), jnp.float32)

    def zrow(i, carry):
        for j in range(DP // 16):
            rows[2][i, pl.ds(j * 16, 16)] = zv
        return carry

    lax.fori_loop(0, CHUNK, zrow, 0)
    rem = ROWS_PT - (ROWS_PT // CHUNK) * CHUNK
    zcopies = [
        pltpu.make_async_copy(
            rows[2], acc.at[pl.ds(s * ROWS_PT + j * CHUNK, CHUNK)], zsem)
        for j in range(ROWS_PT // CHUNK)
    ] + [
        pltpu.make_async_copy(
            rows[2].at[pl.ds(0, rem)],
            acc.at[pl.ds(s * ROWS_PT + ROWS_PT - rem, rem)], zsem)
    ]
    for zc in zcopies:
        zc.start()
    for zc in zcopies:
        zc.wait()
    plsc.subcore_barrier()

    def pipeline(n):
        def body(jj, carry):
            for u in range(K):
                b = u % K
                j = jj * K + u
                gather(j, b).wait()
                scatter(j, b).start(add=True)
                b2 = (u + 2) % K

                @pl.when(j + 2 < n)
                def _():
                    @pl.when(j >= 1)
                    def _():
                        scatter(j - 1, b2).wait()
                    unpack(j + 2, b2)
                    gather(j + 2, b2).start()

            return carry

        lax.fori_loop(0, n // K, body, 0)
        for u in range(K):
            j = n - K + u
            scatter(j, j % K).wait()

    @pl.when(c == 0)
    def _():
        pipeline(A)

    @pl.when(c == 1)
    def _():
        pipeline(B)

    plsc.subcore_barrier()

    off = s * ROWS_PT
    pltpu.sync_copy(acc.at[pl.ds(off, ROWS_PT)], out.at[c, pl.ds(off, ROWS_PT)])


BN = 400


def _finish_body(p_ref, x_ref, o_ref):
    p0 = p_ref[0]
    p1 = p_ref[1]
    cnt = p0[:, D:D + 1] + p1[:, D:D + 1]
    sums = p0[:, :D] + p1[:, :D]
    o_ref[:, :D] = sums / jnp.maximum(cnt, 1.0)
    o_ref[:, D:] = jnp.where(cnt > 0.0, x_ref[...], 0.0)


_finish = pl.pallas_call(
    _finish_body,
    grid=(N // BN,),
    in_specs=[
        pl.BlockSpec((2, BN, DP), lambda i: (0, i, 0)),
        pl.BlockSpec((BN, D), lambda i: (i, 0)),
    ],
    out_specs=pl.BlockSpec((BN, 2 * D), lambda i: (i, 0)),
    out_shape=jax.ShapeDtypeStruct((N, 2 * D), jnp.float32),
)


@jax.jit
def kernel(x, es):
    col = es[0]
    row = es[1]
    xext = jnp.concatenate(
        [x,
         jnp.ones((N, 1), jnp.float32),
         jnp.zeros((N, DP - D - 1), jnp.float32)], axis=1)
    pad = E_PAD - E
    rowp = jnp.concatenate([row, jnp.zeros((pad,), jnp.int32)])
    colp = jnp.concatenate([col, jnp.full((pad,), TRASH, jnp.int32)])
    packed = jnp.bitwise_or(jnp.left_shift(colp, 16), rowp)
    partial = _sc_accumulate(xext, packed.reshape(T_CHUNKS, CHUNK))
    return _finish(partial, x)
